# baseline (speedup 1.0000x reference)
import jax
import jax.numpy as jnp
from jax.experimental import pallas as pl
from jax.experimental.pallas import tpu as pltpu

_E = 8
_CAP = 320
_S = 2048
_D = 1024
_F = 2048
_B = 2
_FSPLIT = 4
_FBLK = _F // _FSPLIT

_bf16 = jnp.bfloat16


def _router_body(x_ref, gw_ref, logits_ref, pmax_ref, keep_ref, eidx_ref,
                 terow_ref, psrow_ref, xb_ref, xlo_ref):
    x = x_ref[0]
    xb = x.astype(_bf16)
    xb_ref[0] = xb
    xlo_ref[0] = (x - xb.astype(jnp.float32)).astype(_bf16)
    gw = gw_ref[...]
    l = jax.lax.dot_general(x, gw, (((1,), (1,)), ((), ())),
                            preferred_element_type=jnp.float32)
    logits_ref[0] = l
    m = jnp.max(l, axis=0, keepdims=True)
    u = jnp.exp(l - m)
    z = jnp.sum(u, axis=0, keepdims=True)
    probs = u / z
    best = probs[:, 0:1]
    te_f = jnp.zeros((_S, 1), jnp.float32)
    for e in range(1, _E):
        pe = probs[:, e:e + 1]
        gt = pe > best
        te_f = jnp.where(gt, jnp.float32(e), te_f)
        best = jnp.where(gt, pe, best)
    pmax_ref[0] = best
    iota_e = jax.lax.broadcasted_iota(jnp.int32, (_S, _E), 1).astype(
        jnp.float32)
    oh = (iota_e == te_f).astype(jnp.float32)
    r = jax.lax.broadcasted_iota(jnp.int32, (128, 128), 0)
    c = jax.lax.broadcasted_iota(jnp.int32, (128, 128), 1)
    tri = (r >= c).astype(jnp.float32)
    eye = (r == c).astype(jnp.float32)
    carry = jnp.zeros((1, _E), jnp.float32)
    sel_cols = []
    te_rows = []
    ps_rows = []
    for k in range(_S // 128):
        sl = slice(k * 128, (k + 1) * 128)
        blk = oh[sl, :]
        pb = jax.lax.dot_general(tri, blk, (((1,), (0,)), ((), ())),
                                 precision=jax.lax.Precision.HIGHEST,
                                 preferred_element_type=jnp.float32) + carry
        carry = pb[127:128, :]
        sel_blk = jnp.sum(blk * pb, axis=1, keepdims=True)
        sel_cols.append(sel_blk)
        te_rows.append(jax.lax.dot_general(
            te_f[sl, :], eye, (((0,), (0,)), ((), ())),
            precision=jax.lax.Precision.HIGHEST,
            preferred_element_type=jnp.float32))
        ps_rows.append(jax.lax.dot_general(
            sel_blk, eye, (((0,), (0,)), ((), ())),
            precision=jax.lax.Precision.HIGHEST,
            preferred_element_type=jnp.float32))
    prio_sel = jnp.concatenate(sel_cols, axis=0)
    keep = (prio_sel <= _CAP).astype(jnp.float32)
    keep_ref[0] = keep
    eidx_ref[0] = (te_f * keep).astype(jnp.int32)
    terow_ref[0] = jnp.concatenate(te_rows, axis=1)
    psrow_ref[0] = jnp.concatenate(ps_rows, axis=1)


def _expert_body(xb_ref, xlo_ref, w1_ref, w2_ref, terow_ref, psrow_ref,
                 pmax_ref, keep_ref, out_ref, msk_ref, xe_ref, y_ref):
    e = pl.program_id(1)
    f = pl.program_id(2)

    @pl.when(f == 0)
    def _():
        te_row = terow_ref[0]
        ps_row = psrow_ref[0]
        cio = (jax.lax.broadcasted_iota(jnp.int32, (_CAP, _S), 0) + 1
               ).astype(jnp.float32)
        msk_ref[...] = ((te_row == e.astype(jnp.float32)) & (ps_row == cio)
                        ).astype(_bf16)
        xe_ref[...] = jax.lax.dot_general(
            msk_ref[...], xb_ref[0], (((1,), (0,)), ((), ())),
            preferred_element_type=jnp.float32).astype(_bf16)

    h = jnp.maximum(
        jax.lax.dot_general(xe_ref[...], w1_ref[0], (((1,), (0,)), ((), ())),
                            preferred_element_type=jnp.float32), 0.0)
    yp = jax.lax.dot_general(h.astype(_bf16), w2_ref[0],
                             (((1,), (0,)), ((), ())),
                             preferred_element_type=jnp.float32)

    @pl.when(f == 0)
    def _():
        y_ref[...] = yp

    @pl.when(f > 0)
    def _():
        y_ref[...] = y_ref[...] + yp

    @pl.when((e == 0) & (f == 0))
    def _():
        out_ref[0] = (1.0 - keep_ref[0]) * (
            xb_ref[0].astype(jnp.float32) + xlo_ref[0].astype(jnp.float32))

    @pl.when(f == _FSPLIT - 1)
    def _():
        y = y_ref[...]
        yhi = y.astype(_bf16)
        ylo = (y - yhi.astype(jnp.float32)).astype(_bf16)
        out_ref[0] = out_ref[0] + jax.lax.dot_general(
            msk_ref[...], yhi, (((0,), (0,)), ((), ())),
            preferred_element_type=jnp.float32) + jax.lax.dot_general(
            msk_ref[...], ylo, (((0,), (0,)), ((), ())),
            preferred_element_type=jnp.float32)

    @pl.when((e == _E - 1) & (f == _FSPLIT - 1))
    def _():
        out_ref[0] = pmax_ref[0] * out_ref[0]


def kernel(norm_data, gate_w, W1, W2):
    f32 = jnp.float32
    i32 = jnp.int32
    (logits, pmax, keep, eidx, te_row, ps_row, xb, xlo) = pl.pallas_call(
        _router_body,
        grid=(_B,),
        in_specs=[
            pl.BlockSpec((1, _S, _D), lambda b: (b, 0, 0)),
            pl.BlockSpec((_E, _D), lambda b: (0, 0)),
        ],
        out_specs=[
            pl.BlockSpec((1, _S, _E), lambda b: (b, 0, 0)),
            pl.BlockSpec((1, _S, 1), lambda b: (b, 0, 0)),
            pl.BlockSpec((1, _S, 1), lambda b: (b, 0, 0)),
            pl.BlockSpec((1, _S, 1), lambda b: (b, 0, 0)),
            pl.BlockSpec((1, 1, _S), lambda b: (b, 0, 0)),
            pl.BlockSpec((1, 1, _S), lambda b: (b, 0, 0)),
            pl.BlockSpec((1, _S, _D), lambda b: (b, 0, 0)),
            pl.BlockSpec((1, _S, _D), lambda b: (b, 0, 0)),
        ],
        out_shape=[
            jax.ShapeDtypeStruct((_B, _S, _E), f32),
            jax.ShapeDtypeStruct((_B, _S, 1), f32),
            jax.ShapeDtypeStruct((_B, _S, 1), f32),
            jax.ShapeDtypeStruct((_B, _S, 1), i32),
            jax.ShapeDtypeStruct((_B, 1, _S), f32),
            jax.ShapeDtypeStruct((_B, 1, _S), f32),
            jax.ShapeDtypeStruct((_B, _S, _D), _bf16),
            jax.ShapeDtypeStruct((_B, _S, _D), _bf16),
        ],
    )(norm_data, gate_w)

    hidden = pl.pallas_call(
        _expert_body,
        grid=(_B, _E, _FSPLIT),
        in_specs=[
            pl.BlockSpec((1, _S, _D), lambda b, e, f: (b, 0, 0)),
            pl.BlockSpec((1, _S, _D), lambda b, e, f: (b, 0, 0)),
            pl.BlockSpec((1, _D, _FBLK), lambda b, e, f: (e, 0, f)),
            pl.BlockSpec((1, _FBLK, _D), lambda b, e, f: (e, f, 0)),
            pl.BlockSpec((1, 1, _S), lambda b, e, f: (b, 0, 0)),
            pl.BlockSpec((1, 1, _S), lambda b, e, f: (b, 0, 0)),
            pl.BlockSpec((1, _S, 1), lambda b, e, f: (b, 0, 0)),
            pl.BlockSpec((1, _S, 1), lambda b, e, f: (b, 0, 0)),
        ],
        out_specs=pl.BlockSpec((1, _S, _D), lambda b, e, f: (b, 0, 0)),
        out_shape=jax.ShapeDtypeStruct((_B, _S, _D), f32),
        scratch_shapes=[
            pltpu.VMEM((_CAP, _S), _bf16),
            pltpu.VMEM((_CAP, _D), _bf16),
            pltpu.VMEM((_CAP, _D), f32),
        ],
        compiler_params=pltpu.CompilerParams(
            dimension_semantics=("arbitrary", "arbitrary", "arbitrary")),
    )(xb, xlo, W1.astype(_bf16), W2.astype(_bf16),
      te_row, ps_row, pmax, keep)

    return hidden, logits, eidx.reshape(_B, _S)


# speedup vs baseline: 1.4371x; 1.4371x over previous
import jax
import jax.numpy as jnp
from jax.experimental import pallas as pl
from jax.experimental.pallas import tpu as pltpu

_E = 8
_CAP = 320
_S = 2048
_D = 1024
_F = 2048
_B = 2
_FSPLIT = 4
_FBLK = _F // _FSPLIT


def _router_body(x_ref, gw_ref, logits_ref, pmax_ref, keep_ref, eidx_ref,
                 terow_ref, psrow_ref):
    x = x_ref[0]
    gw = gw_ref[...]
    l = jax.lax.dot_general(x, gw, (((1,), (1,)), ((), ())),
                            preferred_element_type=jnp.float32)
    logits_ref[0] = l
    m = jnp.max(l, axis=0, keepdims=True)
    u = jnp.exp(l - m)
    z = jnp.sum(u, axis=0, keepdims=True)
    probs = u / z
    best = probs[:, 0:1]
    te_f = jnp.zeros((_S, 1), jnp.float32)
    for e in range(1, _E):
        pe = probs[:, e:e + 1]
        gt = pe > best
        te_f = jnp.where(gt, jnp.float32(e), te_f)
        best = jnp.where(gt, pe, best)
    pmax_ref[0] = best
    iota_e = jax.lax.broadcasted_iota(jnp.int32, (_S, _E), 1).astype(
        jnp.float32)
    oh = (iota_e == te_f).astype(jnp.float32)
    r = jax.lax.broadcasted_iota(jnp.int32, (128, 128), 0)
    c = jax.lax.broadcasted_iota(jnp.int32, (128, 128), 1)
    tri = (r >= c).astype(jnp.float32)
    eye = (r == c).astype(jnp.float32)
    carry = jnp.zeros((1, _E), jnp.float32)
    sel_cols = []
    te_rows = []
    ps_rows = []
    for k in range(_S // 128):
        sl = slice(k * 128, (k + 1) * 128)
        blk = oh[sl, :]
        pb = jax.lax.dot_general(tri, blk, (((1,), (0,)), ((), ())),
                                 precision=jax.lax.Precision.HIGHEST,
                                 preferred_element_type=jnp.float32) + carry
        carry = pb[127:128, :]
        sel_blk = jnp.sum(blk * pb, axis=1, keepdims=True)
        sel_cols.append(sel_blk)
        te_rows.append(jax.lax.dot_general(
            te_f[sl, :], eye, (((0,), (0,)), ((), ())),
            precision=jax.lax.Precision.HIGHEST,
            preferred_element_type=jnp.float32))
        ps_rows.append(jax.lax.dot_general(
            sel_blk, eye, (((0,), (0,)), ((), ())),
            precision=jax.lax.Precision.HIGHEST,
            preferred_element_type=jnp.float32))
    prio_sel = jnp.concatenate(sel_cols, axis=0)
    keep = (prio_sel <= _CAP).astype(jnp.float32)
    keep_ref[0] = keep
    eidx_ref[0] = (te_f * keep).astype(jnp.int32)
    terow_ref[0] = jnp.concatenate(te_rows, axis=1)
    psrow_ref[0] = jnp.concatenate(ps_rows, axis=1)


def _expert_body(x_ref, w1_ref, w2_ref, terow_ref, psrow_ref, pmax_ref,
                 keep_ref, out_ref, msk_ref, xe_ref, y_ref):
    e = pl.program_id(1)
    f = pl.program_id(2)

    @pl.when(f == 0)
    def _():
        te_row = terow_ref[0]
        ps_row = psrow_ref[0]
        cio = (jax.lax.broadcasted_iota(jnp.int32, (_CAP, _S), 0) + 1
               ).astype(jnp.float32)
        msk_ref[...] = ((te_row == e.astype(jnp.float32)) & (ps_row == cio)
                        ).astype(jnp.float32)
        xe_ref[...] = jax.lax.dot_general(
            msk_ref[...], x_ref[0], (((1,), (0,)), ((), ())),
            preferred_element_type=jnp.float32)

    h = jnp.maximum(
        jax.lax.dot_general(xe_ref[...], w1_ref[0], (((1,), (0,)), ((), ())),
                            preferred_element_type=jnp.float32), 0.0)
    yp = jax.lax.dot_general(h, w2_ref[0], (((1,), (0,)), ((), ())),
                             preferred_element_type=jnp.float32)

    @pl.when(f == 0)
    def _():
        y_ref[...] = yp

    @pl.when(f > 0)
    def _():
        y_ref[...] = y_ref[...] + yp

    @pl.when((e == 0) & (f == 0))
    def _():
        out_ref[0] = (1.0 - keep_ref[0]) * x_ref[0]

    @pl.when(f == _FSPLIT - 1)
    def _():
        out_ref[0] = out_ref[0] + jax.lax.dot_general(
            msk_ref[...], y_ref[...], (((0,), (0,)), ((), ())),
            preferred_element_type=jnp.float32)

    @pl.when((e == _E - 1) & (f == _FSPLIT - 1))
    def _():
        out_ref[0] = pmax_ref[0] * out_ref[0]


def kernel(norm_data, gate_w, W1, W2):
    f32 = jnp.float32
    i32 = jnp.int32
    logits, pmax, keep, eidx, te_row, ps_row = pl.pallas_call(
        _router_body,
        grid=(_B,),
        in_specs=[
            pl.BlockSpec((1, _S, _D), lambda b: (b, 0, 0)),
            pl.BlockSpec((_E, _D), lambda b: (0, 0)),
        ],
        out_specs=[
            pl.BlockSpec((1, _S, _E), lambda b: (b, 0, 0)),
            pl.BlockSpec((1, _S, 1), lambda b: (b, 0, 0)),
            pl.BlockSpec((1, _S, 1), lambda b: (b, 0, 0)),
            pl.BlockSpec((1, _S, 1), lambda b: (b, 0, 0)),
            pl.BlockSpec((1, 1, _S), lambda b: (b, 0, 0)),
            pl.BlockSpec((1, 1, _S), lambda b: (b, 0, 0)),
        ],
        out_shape=[
            jax.ShapeDtypeStruct((_B, _S, _E), f32),
            jax.ShapeDtypeStruct((_B, _S, 1), f32),
            jax.ShapeDtypeStruct((_B, _S, 1), f32),
            jax.ShapeDtypeStruct((_B, _S, 1), i32),
            jax.ShapeDtypeStruct((_B, 1, _S), f32),
            jax.ShapeDtypeStruct((_B, 1, _S), f32),
        ],
    )(norm_data, gate_w)

    hidden = pl.pallas_call(
        _expert_body,
        grid=(_B, _E, _FSPLIT),
        in_specs=[
            pl.BlockSpec((1, _S, _D), lambda b, e, f: (b, 0, 0)),
            pl.BlockSpec((1, _D, _FBLK), lambda b, e, f: (e, 0, f)),
            pl.BlockSpec((1, _FBLK, _D), lambda b, e, f: (e, f, 0)),
            pl.BlockSpec((1, 1, _S), lambda b, e, f: (b, 0, 0)),
            pl.BlockSpec((1, 1, _S), lambda b, e, f: (b, 0, 0)),
            pl.BlockSpec((1, _S, 1), lambda b, e, f: (b, 0, 0)),
            pl.BlockSpec((1, _S, 1), lambda b, e, f: (b, 0, 0)),
        ],
        out_specs=pl.BlockSpec((1, _S, _D), lambda b, e, f: (b, 0, 0)),
        out_shape=jax.ShapeDtypeStruct((_B, _S, _D), f32),
        scratch_shapes=[
            pltpu.VMEM((_CAP, _S), f32),
            pltpu.VMEM((_CAP, _D), f32),
            pltpu.VMEM((_CAP, _D), f32),
        ],
        compiler_params=pltpu.CompilerParams(
            dimension_semantics=("arbitrary", "arbitrary", "arbitrary")),
    )(norm_data, W1, W2, te_row, ps_row, pmax, keep)

    return hidden, logits, eidx.reshape(_B, _S)
